# baseline (device time: 32872 ns/iter reference)
import jax
import jax.numpy as jnp
from jax import lax
from jax.experimental import pallas as pl
from jax.experimental.pallas import tpu as pltpu

N_DEV = 4
T = 512
D = 1024
V_LOC = 8192
V_TILE = 2048
N_TILES = V_LOC // V_TILE
K_HALF = D // 2


def kernel(x, W, labels):
    labels2d = labels.reshape(T, 1)

    def body(x_ref, wa_ref, wb_ref, lab_ref, out_ref,
             comm_ref, acc_s, acc_c, send_sems, recv_sems):
        my_i = lax.axis_index("i")
        t = pl.program_id(0)
        barrier_sem = pltpu.get_barrier_semaphore()

        @pl.when(t == 0)
        def _():
            for d in range(1, N_DEV):
                pl.semaphore_signal(
                    barrier_sem, inc=1,
                    device_id=((my_i + d) % N_DEV,),
                    device_id_type=pl.DeviceIdType.MESH,
                )
            pl.semaphore_wait(barrier_sem, N_DEV - 1)

        xb = x_ref[...].astype(jnp.bfloat16)
        wa = wa_ref[...].astype(jnp.bfloat16)
        wb = wb_ref[...].astype(jnp.bfloat16)
        logits = jnp.dot(xb[:, :K_HALF], wa, preferred_element_type=jnp.float32)
        logits = logits + jnp.dot(
            xb[:, K_HALF:], wb, preferred_element_type=jnp.float32
        )

        exp_l = jnp.exp(logits)
        tgt = lab_ref[...] - my_i * V_LOC
        col = lax.broadcasted_iota(jnp.int32, (T, V_TILE), 1) + t * V_TILE
        masked = jnp.where(col == tgt, logits, 0.0)
        ones = jnp.ones((V_TILE, 128), jnp.float32)
        s_part = jnp.dot(exp_l, ones, preferred_element_type=jnp.float32)
        c_part = jnp.dot(masked, ones, preferred_element_type=jnp.float32)

        is_first = t == 0
        acc_s[...] = jnp.where(is_first, s_part, acc_s[...] + s_part)
        acc_c[...] = jnp.where(is_first, c_part, acc_c[...] + c_part)

        @pl.when(t == N_TILES - 1)
        def _():
            payload = jnp.concatenate(
                [acc_s[:, 0:1], acc_c[:, 0:1],
                 jnp.zeros((T, 6), jnp.float32)], axis=1)
            comm_ref[my_i] = payload

            sends = []
            for d in (2, 1, 3):
                rdma = pltpu.make_async_remote_copy(
                    src_ref=comm_ref.at[my_i],
                    dst_ref=comm_ref.at[my_i],
                    send_sem=send_sems.at[d - 1],
                    recv_sem=recv_sems.at[my_i],
                    device_id=((my_i + d) % N_DEV,),
                    device_id_type=pl.DeviceIdType.MESH,
                )
                rdma.start()
                sends.append(rdma)

            for d in (1, 2, 3):
                src = (my_i - d) % N_DEV
                recv = pltpu.make_async_remote_copy(
                    src_ref=comm_ref.at[src],
                    dst_ref=comm_ref.at[src],
                    send_sem=send_sems.at[d - 1],
                    recv_sem=recv_sems.at[src],
                    device_id=(src,),
                    device_id_type=pl.DeviceIdType.MESH,
                )
                recv.wait_recv()
            for rdma in sends:
                rdma.wait_send()

            stats_all = comm_ref[...]
            s_g = jnp.sum(stats_all[:, :, 0], axis=0)
            c_g = jnp.sum(stats_all[:, :, 1], axis=0)
            out_ref[...] = jnp.log(s_g) - c_g

    return pl.pallas_call(
        body,
        grid=(N_TILES,),
        out_shape=jax.ShapeDtypeStruct((T,), jnp.float32),
        in_specs=[
            pl.BlockSpec((T, D), lambda t: (0, 0)),
            pl.BlockSpec((K_HALF, V_TILE), lambda t: (0, t)),
            pl.BlockSpec((K_HALF, V_TILE), lambda t: (1, t)),
            pl.BlockSpec((T, 1), lambda t: (0, 0)),
        ],
        out_specs=pl.BlockSpec((T,), lambda t: (0,)),
        scratch_shapes=[
            pltpu.VMEM((N_DEV, T, 8), jnp.float32),
            pltpu.VMEM((T, 128), jnp.float32),
            pltpu.VMEM((T, 128), jnp.float32),
            pltpu.SemaphoreType.DMA((N_DEV - 1,)),
            pltpu.SemaphoreType.DMA((N_DEV,)),
        ],
        compiler_params=pltpu.CompilerParams(collective_id=0),
    )(x, W, W, labels2d)


# device time: 30767 ns/iter; 1.0684x vs baseline; 1.0684x over previous
import jax
import jax.numpy as jnp
from jax import lax
from jax.experimental import pallas as pl
from jax.experimental.pallas import tpu as pltpu

N_DEV = 4
T = 512
D = 1024
V_LOC = 8192
V_TILE = 2048
N_TILES = V_LOC // V_TILE
K_HALF = D // 2


def kernel(x, W, labels):
    labels2d = labels.reshape(T, 1)

    def body(x_ref, wa_ref, wb_ref, lab_ref, out_ref,
             comm_ref, acc_ref, send_sems, recv_sems):
        my_i = lax.axis_index("i")
        t = pl.program_id(0)
        barrier_sem = pltpu.get_barrier_semaphore()

        @pl.when(t == 0)
        def _():
            for d in range(1, N_DEV):
                pl.semaphore_signal(
                    barrier_sem, inc=1,
                    device_id=((my_i + d) % N_DEV,),
                    device_id_type=pl.DeviceIdType.MESH,
                )
            pl.semaphore_wait(barrier_sem, N_DEV - 1)

        xb = x_ref[...].astype(jnp.bfloat16)
        wa = wa_ref[...].astype(jnp.bfloat16)
        wb = wb_ref[...].astype(jnp.bfloat16)
        logits = jnp.dot(xb[:, :K_HALF], wa, preferred_element_type=jnp.float32)
        logits = logits + jnp.dot(
            xb[:, K_HALF:], wb, preferred_element_type=jnp.float32
        )

        s_t = jnp.sum(jnp.exp(logits), axis=1, keepdims=True)
        tgt = lab_ref[...] - my_i * V_LOC
        col = lax.broadcasted_iota(jnp.int32, (T, V_TILE), 1) + t * V_TILE
        c_t = jnp.sum(jnp.where(col == tgt, logits, 0.0), axis=1,
                      keepdims=True)

        is_first = t == 0
        sc = jnp.concatenate(
            [s_t, c_t, jnp.zeros((T, 6), jnp.float32)], axis=1)
        acc_ref[...] = jnp.where(is_first, sc, acc_ref[...] + sc)

        @pl.when(t == N_TILES - 1)
        def _():
            comm_ref[my_i] = acc_ref[...]

            sends = []
            for d in (2, 1, 3):
                rdma = pltpu.make_async_remote_copy(
                    src_ref=comm_ref.at[my_i],
                    dst_ref=comm_ref.at[my_i],
                    send_sem=send_sems.at[d - 1],
                    recv_sem=recv_sems.at[my_i],
                    device_id=((my_i + d) % N_DEV,),
                    device_id_type=pl.DeviceIdType.MESH,
                )
                rdma.start()
                sends.append(rdma)

            for d in (1, 2, 3):
                src = (my_i - d) % N_DEV
                recv = pltpu.make_async_remote_copy(
                    src_ref=comm_ref.at[src],
                    dst_ref=comm_ref.at[src],
                    send_sem=send_sems.at[d - 1],
                    recv_sem=recv_sems.at[src],
                    device_id=(src,),
                    device_id_type=pl.DeviceIdType.MESH,
                )
                recv.wait_recv()
            for rdma in sends:
                rdma.wait_send()

            stats_all = comm_ref[...]
            s_g = jnp.sum(stats_all[:, :, 0], axis=0)
            c_g = jnp.sum(stats_all[:, :, 1], axis=0)
            out_ref[...] = jnp.log(s_g) - c_g

    return pl.pallas_call(
        body,
        grid=(N_TILES,),
        out_shape=jax.ShapeDtypeStruct((T,), jnp.float32),
        in_specs=[
            pl.BlockSpec((T, D), lambda t: (0, 0)),
            pl.BlockSpec((K_HALF, V_TILE), lambda t: (0, t)),
            pl.BlockSpec((K_HALF, V_TILE), lambda t: (1, t)),
            pl.BlockSpec((T, 1), lambda t: (0, 0)),
        ],
        out_specs=pl.BlockSpec((T,), lambda t: (0,)),
        scratch_shapes=[
            pltpu.VMEM((N_DEV, T, 8), jnp.float32),
            pltpu.VMEM((T, 8), jnp.float32),
            pltpu.SemaphoreType.DMA((N_DEV - 1,)),
            pltpu.SemaphoreType.DMA((N_DEV,)),
        ],
        compiler_params=pltpu.CompilerParams(collective_id=0),
    )(x, W, W, labels2d)


# device time: 23962 ns/iter; 1.3718x vs baseline; 1.2840x over previous
import jax
import jax.numpy as jnp
from jax import lax
from jax.experimental import pallas as pl
from jax.experimental.pallas import tpu as pltpu

N_DEV = 4
T = 512
D = 1024
V_LOC = 8192
V_TILE = 2048
N_TILES = V_LOC // V_TILE
K_HALF = D // 2


def kernel(x, W, labels):
    labels2d = labels.reshape(1, T)

    def body(x_ref, wa_ref, wb_ref, lab_ref, out_ref,
             comm_ref, send_sems, recv_sems):
        my_i = lax.axis_index("i")
        t = pl.program_id(0)
        barrier_sem = pltpu.get_barrier_semaphore()

        @pl.when(t == 0)
        def _():
            for d in range(1, N_DEV):
                pl.semaphore_signal(
                    barrier_sem, inc=1,
                    device_id=((my_i + d) % N_DEV,),
                    device_id_type=pl.DeviceIdType.MESH,
                )
            pl.semaphore_wait(barrier_sem, N_DEV - 1)

        xb = x_ref[...].astype(jnp.bfloat16)
        wa = wa_ref[...].astype(jnp.bfloat16)
        wb = wb_ref[...].astype(jnp.bfloat16)
        logits = jnp.dot(xb[:, :K_HALF], wa, preferred_element_type=jnp.float32)
        logits = logits + jnp.dot(
            xb[:, K_HALF:], wb, preferred_element_type=jnp.float32
        )

        lt = logits.T
        s_t = jnp.sum(jnp.exp(lt), axis=0)
        tgt = lab_ref[...] - my_i * V_LOC
        row = lax.broadcasted_iota(jnp.int32, (V_TILE, T), 0) + t * V_TILE
        c_t = jnp.sum(jnp.where(row == tgt, lt, 0.0), axis=0)

        comm_ref[my_i, t, :] = s_t
        comm_ref[my_i, N_TILES + t, :] = c_t

        @pl.when(t == N_TILES - 1)
        def _():
            sends = []
            for d in (2, 1, 3):
                rdma = pltpu.make_async_remote_copy(
                    src_ref=comm_ref.at[my_i],
                    dst_ref=comm_ref.at[my_i],
                    send_sem=send_sems.at[d - 1],
                    recv_sem=recv_sems.at[my_i],
                    device_id=((my_i + d) % N_DEV,),
                    device_id_type=pl.DeviceIdType.MESH,
                )
                rdma.start()
                sends.append(rdma)

            for d in (1, 2, 3):
                src = (my_i - d) % N_DEV
                recv = pltpu.make_async_remote_copy(
                    src_ref=comm_ref.at[src],
                    dst_ref=comm_ref.at[src],
                    send_sem=send_sems.at[d - 1],
                    recv_sem=recv_sems.at[src],
                    device_id=(src,),
                    device_id_type=pl.DeviceIdType.MESH,
                )
                recv.wait_recv()
            for rdma in sends:
                rdma.wait_send()

            stats_all = comm_ref[...]
            s_g = jnp.sum(stats_all[:, :N_TILES, :], axis=(0, 1))
            c_g = jnp.sum(stats_all[:, N_TILES:, :], axis=(0, 1))
            out_ref[...] = jnp.log(s_g) - c_g

    return pl.pallas_call(
        body,
        grid=(N_TILES,),
        out_shape=jax.ShapeDtypeStruct((T,), jnp.float32),
        in_specs=[
            pl.BlockSpec((T, D), lambda t: (0, 0)),
            pl.BlockSpec((K_HALF, V_TILE), lambda t: (0, t)),
            pl.BlockSpec((K_HALF, V_TILE), lambda t: (1, t)),
            pl.BlockSpec((1, T), lambda t: (0, 0)),
        ],
        out_specs=pl.BlockSpec((T,), lambda t: (0,)),
        scratch_shapes=[
            pltpu.VMEM((N_DEV, 2 * N_TILES, T), jnp.float32),
            pltpu.SemaphoreType.DMA((N_DEV - 1,)),
            pltpu.SemaphoreType.DMA((N_DEV,)),
        ],
        compiler_params=pltpu.CompilerParams(collective_id=0),
    )(x, W, W, labels2d)
